# baseline (speedup 1.0000x reference)
import jax
import jax.numpy as jnp
from jax.experimental import pallas as pl

_TEMPERATURE = 1.0
_TOP_P = 0.9
_TOP_K = 64


def _tail_kernel(vals_ref, idx_ref, gumbel_ref, k_ref, tok_ref, probs_ref):
    vals = vals_ref[...]
    idx = idx_ref[...]
    gumbel = gumbel_ref[...]
    k = k_ref[0]
    probs = jax.nn.softmax(vals, axis=-1)
    tri = (jax.lax.broadcasted_iota(jnp.int32, (_TOP_K, _TOP_K), 0)
           >= jax.lax.broadcasted_iota(jnp.int32, (_TOP_K, _TOP_K), 1))
    cum = jax.lax.dot_general(probs, tri.astype(jnp.float32),
                              (((1,), (1,)), ((), ())),
                              preferred_element_type=jnp.float32)
    lane = jax.lax.broadcasted_iota(jnp.int32, vals.shape, 1)
    keep = (cum <= _TOP_P) | (lane == 0)
    keep = keep & (lane < k)
    masked = jnp.where(keep, vals, -jnp.inf)
    score = masked + gumbel
    best = jnp.max(score, axis=-1, keepdims=True)
    choice = jnp.min(jnp.where(score == best, lane, _TOP_K), axis=-1,
                     keepdims=True)
    tok_ref[...] = jnp.sum(jnp.where(lane == choice, idx, 0), axis=-1)
    probs_ref[...] = jax.nn.softmax(masked, axis=-1)


def kernel(logits, k):
    preds = logits / _TEMPERATURE
    topk_vals, topk_idx = jax.lax.top_k(preds, _TOP_K)
    gkey = jax.random.key(42)
    gumbel = jax.random.gumbel(gkey, topk_vals.shape, dtype=topk_vals.dtype)
    k_arr = jnp.asarray(k, jnp.int32).reshape((1,))
    tok, probs = pl.pallas_call(
        _tail_kernel,
        out_shape=(
            jax.ShapeDtypeStruct((logits.shape[0],), jnp.int32),
            jax.ShapeDtypeStruct(topk_vals.shape, jnp.float32),
        ),
    )(topk_vals, topk_idx, gumbel, k_arr)
    return tok, probs


# speedup vs baseline: 1.5656x; 1.5656x over previous
import functools

import jax
import jax.numpy as jnp
from jax import lax
from jax.experimental import pallas as pl
from jax.experimental.pallas import tpu as pltpu
from jax.experimental.pallas import tpu_sc as plsc

_TOP_P = 0.9
_TOP_K = 64

_R = 128
_V = 100000
_NW = 32
_ROWS_PER_W = _R // _NW
_CHUNKS = _V // 16

_NBINS = 1024
_HIST_SLOTS = 16 * _NBINS

_HI_REGION = 96
_CAP_IN = 240
_BUF = _HI_REGION + _CAP_IN + 16
_NVREG = _BUF // 16

_BIG = 1 << 30


def _sc_body(logits_hbm, gumbel_hbm, kinf_hbm, tok_hbm, probs_hbm,
             row_v, hist, buf_v, buf_i, out_v, out_i, g_v, kinf_v,
             probs_st, tok_st):
    wid = lax.axis_index("s") * 2 + lax.axis_index("c")
    iota = lax.iota(jnp.int32, 16)
    lane0 = iota == 0
    zeros_i = jnp.zeros((16,), jnp.int32)
    ones_i = jnp.ones((16,), jnp.int32)
    neginf = jnp.full((16,), -jnp.inf, jnp.float32)
    lane_base = iota * jnp.int32(_NBINS)

    pltpu.sync_copy(kinf_hbm, kinf_v)

    def row_body(r, _):
        row = wid * _ROWS_PER_W + r
        pltpu.sync_copy(logits_hbm.at[row], row_v)
        pltpu.sync_copy(gumbel_hbm.at[row], g_v)

        def zh(z, _c):
            hist[pl.ds(z * 16, 16)] = zeros_i
            return 0
        lax.fori_loop(0, _HIST_SLOTS // 16, zh, 0)

        def zb(z, _c):
            buf_v[pl.ds(z * 16, 16)] = neginf
            return 0
        lax.fori_loop(0, _NVREG, zb, 0)

        def mm_body(c, carry):
            nlo_v, hi_v = carry
            v = row_v[pl.ds(c * 16, 16)]
            return (jnp.maximum(nlo_v, -v), jnp.maximum(hi_v, v))
        nlo_v, hi_v = lax.fori_loop(0, _CHUNKS, mm_body, (neginf, neginf))
        lo = -jnp.max(nlo_v)
        scale = (jnp.full((16,), _NBINS - 2, jnp.float32)
                 / jnp.broadcast_to(jnp.max(hi_v) - lo, (16,)))

        def to_bin(v):
            ti = ((v - lo) * scale).astype(jnp.int32)
            return jnp.clip(ti, 0, _NBINS - 1)

        def hist_body(c, _c):
            binv = to_bin(row_v[pl.ds(c * 16, 16)])
            plsc.addupdate_scatter(hist, [lane_base + binv], ones_i)
            return 0
        lax.fori_loop(0, _CHUNKS, hist_body, 0)

        def scan_body(i, carry):
            found, bin_b, c_gt, n_b, cum = carry
            cc = 63 - i

            def mbody(s, acc):
                return acc + hist[pl.ds(s * _NBINS + cc * 16, 16)]
            chunk = lax.fori_loop(0, 16, mbody, zeros_i)
            rchunk = lax.rev(chunk, (0,))
            sfx = plsc.cumsum(rchunk) + cum
            ge = sfx >= _TOP_K
            cnt_ge = jnp.sum(ge.astype(jnp.int32))
            pos = jnp.int32(16) - cnt_ge
            s_pos = jnp.min(jnp.where(ge, sfx, jnp.int32(_BIG)))
            prev = jnp.maximum(jnp.max(jnp.where(ge, jnp.int32(-_BIG), sfx)),
                               cum)
            hit = jnp.logical_and(jnp.logical_not(found), cnt_ge > 0)
            return (jnp.logical_or(found, cnt_ge > 0),
                    jnp.where(hit, cc * 16 + 15 - pos, bin_b),
                    jnp.where(hit, prev, c_gt),
                    jnp.where(hit, s_pos - prev, n_b),
                    cum + jnp.sum(rchunk))
        _, bin_b, c_gt, n_b, _ = lax.fori_loop(
            0, 64, scan_body,
            (jnp.bool_(False), jnp.int32(0), jnp.int32(0), jnp.int32(0),
             jnp.int32(0)))

        def col_body(c, carry):
            cnt_hi, cnt_in = carry
            v = row_v[pl.ds(c * 16, 16)]
            binv = to_bin(v)
            mhi = binv > bin_b
            min_ = jnp.logical_and(binv == bin_b, cnt_in < _CAP_IN)
            nhi = jnp.sum(mhi.astype(jnp.int32))
            nin = jnp.sum(min_.astype(jnp.int32))

            @pl.when(nhi + nin > 0)
            def _store():
                idxv = c * 16 + iota
                plsc.store_compressed(buf_v.at[pl.ds(cnt_hi, 16)], v, mask=mhi)
                plsc.store_compressed(buf_i.at[pl.ds(cnt_hi, 16)], idxv, mask=mhi)
                plsc.store_compressed(
                    buf_v.at[pl.ds(_HI_REGION + cnt_in, 16)], v, mask=min_)
                plsc.store_compressed(
                    buf_i.at[pl.ds(_HI_REGION + cnt_in, 16)], idxv, mask=min_)
            return (cnt_hi + nhi, cnt_in + nin)
        lax.fori_loop(0, _CHUNKS, col_body, (jnp.int32(0), jnp.int32(0)))

        def sel_body(i, _c):
            def scan_bufs(j, bc):
                bv, bp = bc
                x = buf_v[pl.ds(j * 16, 16)]
                m = x > bv
                return (jnp.where(m, x, bv),
                        jnp.where(m, j * 16 + iota, bp))
            bv, bp = lax.fori_loop(0, _NVREG, scan_bufs,
                                   (neginf, jnp.full((16,), _BIG, jnp.int32)))
            mx = jnp.max(bv)
            p = jnp.min(jnp.where(bv == mx, bp, jnp.int32(_BIG)))
            pv = jnp.broadcast_to(p, (16,))
            idx_p = jnp.max(plsc.load_gather(buf_i, [pv]))
            plsc.store_scatter(buf_v, [pv], neginf, mask=lane0)
            iv = jnp.broadcast_to(i, (16,))
            plsc.store_scatter(out_v, [iv], jnp.broadcast_to(mx, (16,)),
                               mask=lane0)
            plsc.store_scatter(out_i, [iv], jnp.broadcast_to(idx_p, (16,)),
                               mask=lane0)
            return 0
        lax.fori_loop(0, _TOP_K, sel_body, 0)

        vals = [out_v[pl.ds(j * 16, 16)] for j in range(4)]
        vmax = jnp.max(vals[0])
        es = [jnp.exp(v - vmax) for v in vals]
        s = es[0] + es[1] + es[2] + es[3]
        total = jnp.sum(s)
        keeps, masked = [], []
        carry = jnp.float32(0.0)
        for j in range(4):
            pj = es[j] / total
            cj = plsc.cumsum(pj) + carry
            carry = jnp.max(cj)
            kp = cj <= _TOP_P
            if j == 0:
                kp = jnp.logical_or(kp, lane0)
            keeps.append(kp)
            masked.append(jnp.where(kp, vals[j], -jnp.inf)
                          + kinf_v[pl.ds(j * 16, 16)])
        bm, bp = neginf, jnp.full((16,), _BIG, jnp.int32)
        for j in range(4):
            sc = masked[j] + g_v[pl.ds(j * 16, 16)]
            m = sc > bm
            bm = jnp.where(m, sc, bm)
            bp = jnp.where(m, j * 16 + iota, bp)
        mx2 = jnp.max(bm)
        p2 = jnp.min(jnp.where(bm == mx2, bp, jnp.int32(_BIG)))
        token = jnp.max(plsc.load_gather(out_i, [jnp.broadcast_to(p2, (16,))]))

        e2 = [jnp.where(jnp.logical_and(
                  keeps[j], kinf_v[pl.ds(j * 16, 16)] == 0.0),
                  es[j], 0.0) for j in range(4)]
        s2 = jnp.sum(e2[0] + e2[1] + e2[2] + e2[3])
        for j in range(4):
            probs_st[pl.ds(j * 16, 16)] = e2[j] / s2
        tok_st[...] = jnp.where(lane0, token, 0)

        pltpu.sync_copy(probs_st, probs_hbm.at[row])
        pltpu.sync_copy(tok_st, tok_hbm.at[row])
        return 0

    lax.fori_loop(0, _ROWS_PER_W, row_body, 0)


@functools.partial(jax.jit, static_argnames=())
def _sc_topk_sample(logits, gumbel, kinf):
    mesh = plsc.VectorSubcoreMesh(core_axis_name="c", subcore_axis_name="s")
    f = pl.kernel(
        _sc_body,
        mesh=mesh,
        compiler_params=pltpu.CompilerParams(needs_layout_passes=False),
        out_type=(
            jax.ShapeDtypeStruct((_R, 16), jnp.int32),
            jax.ShapeDtypeStruct((_R, _TOP_K), jnp.float32),
        ),
        scratch_types=[
            pltpu.VMEM((_V,), jnp.float32),
            pltpu.VMEM((_HIST_SLOTS,), jnp.int32),
            pltpu.VMEM((_BUF,), jnp.float32),
            pltpu.VMEM((_BUF,), jnp.int32),
            pltpu.VMEM((_TOP_K,), jnp.float32),
            pltpu.VMEM((_TOP_K,), jnp.int32),
            pltpu.VMEM((_TOP_K,), jnp.float32),
            pltpu.VMEM((_TOP_K,), jnp.float32),
            pltpu.VMEM((_TOP_K,), jnp.float32),
            pltpu.VMEM((16,), jnp.int32),
        ],
    )
    return f(logits, gumbel, kinf)


def kernel(logits, k):
    gumbel = jax.random.gumbel(jax.random.key(42), (_R, _TOP_K), jnp.float32)
    kinf = jnp.where(jnp.arange(_TOP_K) < k, 0.0, -jnp.inf).astype(jnp.float32)
    tok, probs = _sc_topk_sample(logits, gumbel, kinf)
    return tok[:, 0], probs


# speedup vs baseline: 3.2488x; 2.0751x over previous
import functools

import jax
import jax.numpy as jnp
from jax import lax
from jax.experimental import pallas as pl
from jax.experimental.pallas import tpu as pltpu
from jax.experimental.pallas import tpu_sc as plsc

_TOP_P = 0.9
_TOP_K = 64

_R = 128
_V = 100000
_NW = 32
_ROWS_PER_W = _R // _NW
_CHUNKS = _V // 16

_NBINS = 512
_HIST_SLOTS = 16 * _NBINS

_WCHUNKS = 10
_NWIN = _CHUNKS // _WCHUNKS

_HI_REGION = 96
_CAP_IN = 240
_BUF = _HI_REGION + _CAP_IN + 16
_NVREG = _BUF // 16

_BIG = 1 << 30


def _sc_body(logits_hbm, gumbel_hbm, kinf_hbm, tok_hbm, probs_hbm,
             row_v, hist, wmax, buf_v, buf_i, out_v, out_i, g_v, kinf_v,
             probs_st, tok_st):
    wid = lax.axis_index("s") * 2 + lax.axis_index("c")
    iota = lax.iota(jnp.int32, 16)
    lane0 = iota == 0
    zeros_i = jnp.zeros((16,), jnp.int32)
    ones_i = jnp.ones((16,), jnp.int32)
    neginf = jnp.full((16,), -jnp.inf, jnp.float32)
    lane_base = iota * jnp.int32(_NBINS)

    pltpu.sync_copy(kinf_hbm, kinf_v)

    def row_body(r, _):
        row = wid * _ROWS_PER_W + r
        pltpu.sync_copy(logits_hbm.at[row], row_v)
        pltpu.sync_copy(gumbel_hbm.at[row], g_v)

        def zh(z, _c):
            hist[pl.ds(z * 16, 16)] = zeros_i
            return 0
        lax.fori_loop(0, _HIST_SLOTS // 16, zh, 0)
        for z in range(_NVREG):
            buf_v[pl.ds(z * 16, 16)] = neginf

        def mm_body(c, carry):
            nlo_v, hi_v = carry
            v = row_v[pl.ds(c * 128, 16)]
            return (jnp.maximum(nlo_v, -v), jnp.maximum(hi_v, v))
        nlo_v, hi_v = lax.fori_loop(0, _CHUNKS // 8, mm_body,
                                    (neginf, neginf))
        lo = -jnp.max(nlo_v)
        scale = (jnp.full((16,), _NBINS - 2, jnp.float32)
                 / jnp.broadcast_to(jnp.max(hi_v) - lo, (16,)))

        def to_bin(v):
            ti = ((v - lo) * scale).astype(jnp.int32)
            return jnp.clip(ti, 0, _NBINS - 1)

        def hist_body(w, _c):
            wm = neginf
            for u in range(_WCHUNKS):
                v = row_v[pl.ds((w * _WCHUNKS + u) * 16, 16)]
                plsc.addupdate_scatter(hist, [lane_base + to_bin(v)], ones_i)
                wm = jnp.maximum(wm, v)
            wmax[pl.ds(w * 16, 16)] = wm
            return 0
        lax.fori_loop(0, _NWIN, hist_body, 0)

        def scan_body(i, carry):
            found, bin_b, c_gt, n_b, cum = carry
            cc = (_NBINS // 16 - 1) - i

            def mbody(s, acc):
                return acc + hist[pl.ds(s * _NBINS + cc * 16, 16)]
            chunk = lax.fori_loop(0, 16, mbody, zeros_i)
            rchunk = lax.rev(chunk, (0,))
            sfx = plsc.cumsum(rchunk) + cum
            ge = sfx >= _TOP_K
            cnt_ge = jnp.sum(ge.astype(jnp.int32))
            pos = jnp.int32(16) - cnt_ge
            s_pos = jnp.min(jnp.where(ge, sfx, jnp.int32(_BIG)))
            prev = jnp.maximum(jnp.max(jnp.where(ge, jnp.int32(-_BIG), sfx)),
                               cum)
            hit = jnp.logical_and(jnp.logical_not(found), cnt_ge > 0)
            return (jnp.logical_or(found, cnt_ge > 0),
                    jnp.where(hit, cc * 16 + 15 - pos, bin_b),
                    jnp.where(hit, prev, c_gt),
                    jnp.where(hit, s_pos - prev, n_b),
                    cum + jnp.sum(rchunk))
        _, bin_b, c_gt, n_b, _ = lax.fori_loop(
            0, _NBINS // 16, scan_body,
            (jnp.bool_(False), jnp.int32(0), jnp.int32(0), jnp.int32(0),
             jnp.int32(0)))

        lf_v = (lo + (jnp.broadcast_to(bin_b, (16,)).astype(jnp.float32)
                      - 2.0) / scale)

        def col_window(w, carry):
            hit = jnp.any(wmax[pl.ds(w * 16, 16)] >= lf_v)

            def do(carry):
                cnt_hi, cnt_in = carry
                for u in range(_WCHUNKS):
                    c = w * _WCHUNKS + u
                    v = row_v[pl.ds(c * 16, 16)]
                    binv = to_bin(v)
                    mhi = binv > bin_b
                    min_ = jnp.logical_and(binv == bin_b, cnt_in < _CAP_IN)
                    nhi = jnp.sum(mhi.astype(jnp.int32))
                    nin = jnp.sum(min_.astype(jnp.int32))

                    @pl.when(nhi + nin > 0)
                    def _store(cnt_hi=cnt_hi, cnt_in=cnt_in, v=v,
                               mhi=mhi, min_=min_, c=c):
                        idxv = c * 16 + iota
                        plsc.store_compressed(buf_v.at[pl.ds(cnt_hi, 16)],
                                              v, mask=mhi)
                        plsc.store_compressed(buf_i.at[pl.ds(cnt_hi, 16)],
                                              idxv, mask=mhi)
                        plsc.store_compressed(
                            buf_v.at[pl.ds(_HI_REGION + cnt_in, 16)],
                            v, mask=min_)
                        plsc.store_compressed(
                            buf_i.at[pl.ds(_HI_REGION + cnt_in, 16)],
                            idxv, mask=min_)
                    cnt_hi = cnt_hi + nhi
                    cnt_in = cnt_in + nin
                return (cnt_hi, cnt_in)

            return lax.cond(hit, do, lambda c_: c_, carry)
        lax.fori_loop(0, _NWIN, col_window, (jnp.int32(0), jnp.int32(0)))

        def sel_body(i, _c):
            bv, bp = neginf, jnp.full((16,), _BIG, jnp.int32)
            for j in range(_NVREG):
                x = buf_v[pl.ds(j * 16, 16)]
                m = x > bv
                bv = jnp.where(m, x, bv)
                bp = jnp.where(m, j * 16 + iota, bp)
            mx = jnp.max(bv)
            p = jnp.min(jnp.where(bv == mx, bp, jnp.int32(_BIG)))
            pv = jnp.broadcast_to(p, (16,))
            idx_p = jnp.max(plsc.load_gather(buf_i, [pv]))
            plsc.store_scatter(buf_v, [pv], neginf, mask=lane0)
            iv = jnp.broadcast_to(i, (16,))
            plsc.store_scatter(out_v, [iv], jnp.broadcast_to(mx, (16,)),
                               mask=lane0)
            plsc.store_scatter(out_i, [iv], jnp.broadcast_to(idx_p, (16,)),
                               mask=lane0)
            return 0
        lax.fori_loop(0, _TOP_K, sel_body, 0)

        vals = [out_v[pl.ds(j * 16, 16)] for j in range(4)]
        vmax = jnp.max(vals[0])
        es = [jnp.exp(v - vmax) for v in vals]
        s = es[0] + es[1] + es[2] + es[3]
        total = jnp.sum(s)
        keeps, masked = [], []
        carry = jnp.float32(0.0)
        for j in range(4):
            pj = es[j] / total
            cj = plsc.cumsum(pj) + carry
            carry = jnp.max(cj)
            kp = cj <= _TOP_P
            if j == 0:
                kp = jnp.logical_or(kp, lane0)
            keeps.append(kp)
            masked.append(jnp.where(kp, vals[j], -jnp.inf)
                          + kinf_v[pl.ds(j * 16, 16)])
        bm, bp = neginf, jnp.full((16,), _BIG, jnp.int32)
        for j in range(4):
            sc = masked[j] + g_v[pl.ds(j * 16, 16)]
            m = sc > bm
            bm = jnp.where(m, sc, bm)
            bp = jnp.where(m, j * 16 + iota, bp)
        mx2 = jnp.max(bm)
        p2 = jnp.min(jnp.where(bm == mx2, bp, jnp.int32(_BIG)))
        token = jnp.max(plsc.load_gather(out_i, [jnp.broadcast_to(p2, (16,))]))

        e2 = [jnp.where(jnp.logical_and(
                  keeps[j], kinf_v[pl.ds(j * 16, 16)] == 0.0),
                  es[j], 0.0) for j in range(4)]
        s2 = jnp.sum(e2[0] + e2[1] + e2[2] + e2[3])
        for j in range(4):
            probs_st[pl.ds(j * 16, 16)] = e2[j] / s2
        tok_st[...] = jnp.where(lane0, token, 0)

        pltpu.sync_copy(probs_st, probs_hbm.at[row])
        pltpu.sync_copy(tok_st, tok_hbm.at[row])
        return 0

    lax.fori_loop(0, _ROWS_PER_W, row_body, 0)


@functools.partial(jax.jit, static_argnames=())
def _sc_topk_sample(logits, gumbel, kinf):
    mesh = plsc.VectorSubcoreMesh(core_axis_name="c", subcore_axis_name="s")
    f = pl.kernel(
        _sc_body,
        mesh=mesh,
        compiler_params=pltpu.CompilerParams(needs_layout_passes=False),
        out_type=(
            jax.ShapeDtypeStruct((_R, 16), jnp.int32),
            jax.ShapeDtypeStruct((_R, _TOP_K), jnp.float32),
        ),
        scratch_types=[
            pltpu.VMEM((_V,), jnp.float32),
            pltpu.VMEM((_HIST_SLOTS,), jnp.int32),
            pltpu.VMEM((_NWIN * 16,), jnp.float32),
            pltpu.VMEM((_BUF,), jnp.float32),
            pltpu.VMEM((_BUF,), jnp.int32),
            pltpu.VMEM((_TOP_K,), jnp.float32),
            pltpu.VMEM((_TOP_K,), jnp.int32),
            pltpu.VMEM((_TOP_K,), jnp.float32),
            pltpu.VMEM((_TOP_K,), jnp.float32),
            pltpu.VMEM((_TOP_K,), jnp.float32),
            pltpu.VMEM((16,), jnp.int32),
        ],
    )
    return f(logits, gumbel, kinf)


def kernel(logits, k):
    gumbel = jax.random.gumbel(jax.random.key(42), (_R, _TOP_K), jnp.float32)
    kinf = jnp.where(jnp.arange(_TOP_K) < k, 0.0, -jnp.inf).astype(jnp.float32)
    tok, probs = _sc_topk_sample(logits, gumbel, kinf)
    return tok[:, 0], probs


# speedup vs baseline: 3.4091x; 1.0493x over previous
import functools

import jax
import jax.numpy as jnp
from jax import lax
from jax.experimental import pallas as pl
from jax.experimental.pallas import tpu as pltpu
from jax.experimental.pallas import tpu_sc as plsc

_TOP_P = 0.9
_TOP_K = 64

_R = 128
_V = 100000
_NW = 32
_ROWS_PER_W = _R // _NW
_CHUNKS = _V // 16

_NBINS = 512
_HIST_SLOTS = 16 * _NBINS

_WCHUNKS = 10
_NWIN = _CHUNKS // _WCHUNKS

_HI_REGION = 96
_CAP_IN = 240
_BUF = _HI_REGION + _CAP_IN + 16
_NVREG = _BUF // 16

_BIG = 1 << 30


def _sc_body(logits_hbm, gumbel_hbm, kinf_hbm, tok_hbm, probs_hbm,
             row_v, hist, wmax, buf_v, buf_i, out_v, out_i, g_v, kinf_v,
             probs_st, tok_st, dma_sem):
    wid = lax.axis_index("s") * 2 + lax.axis_index("c")
    iota = lax.iota(jnp.int32, 16)
    lane0 = iota == 0
    zeros_i = jnp.zeros((16,), jnp.int32)
    ones_i = jnp.ones((16,), jnp.int32)
    neginf = jnp.full((16,), -jnp.inf, jnp.float32)
    lane_base = iota * jnp.int32(_NBINS)

    pltpu.sync_copy(kinf_hbm, kinf_v)
    first_row = wid * _ROWS_PER_W
    pltpu.async_copy(logits_hbm.at[first_row], row_v, dma_sem)

    def row_body(r, _):
        row = wid * _ROWS_PER_W + r
        pltpu.sync_copy(gumbel_hbm.at[row], g_v)

        def zh(z, _c):
            for u in range(8):
                hist[pl.ds((z * 8 + u) * 16, 16)] = zeros_i
            return 0
        lax.fori_loop(0, _HIST_SLOTS // 128, zh, 0)
        for z in range(_NVREG):
            buf_v[pl.ds(z * 16, 16)] = neginf

        pltpu.make_async_copy(logits_hbm.at[row], row_v, dma_sem).wait()

        def mm_body(c, carry):
            nlo_v, hi_v = carry
            for u in range(4):
                v = row_v[pl.ds((c * 4 + u) * 128, 16)]
                nlo_v = jnp.maximum(nlo_v, -v)
                hi_v = jnp.maximum(hi_v, v)
            return (nlo_v, hi_v)
        nlo_v, hi_v = lax.fori_loop(0, _CHUNKS // 32, mm_body,
                                    (neginf, neginf))
        lo = -jnp.max(nlo_v)
        scale = (jnp.full((16,), _NBINS - 2, jnp.float32)
                 / jnp.broadcast_to(jnp.max(hi_v) - lo, (16,)))

        def to_bin(v):
            ti = ((v - lo) * scale).astype(jnp.int32)
            return jnp.clip(ti, 0, _NBINS - 1)

        def hist_body(w, _c):
            wm = neginf
            for u in range(_WCHUNKS):
                v = row_v[pl.ds((w * _WCHUNKS + u) * 16, 16)]
                plsc.addupdate_scatter(hist, [lane_base + to_bin(v)], ones_i)
                wm = jnp.maximum(wm, v)
            wmax[pl.ds(w * 16, 16)] = wm
            return 0
        lax.fori_loop(0, _NWIN, hist_body, 0)

        def scan_body(i, carry):
            found, bin_b, c_gt, n_b, cum = carry
            cc = (_NBINS // 16 - 1) - i

            chunk = zeros_i
            for sh in range(16):
                chunk = chunk + hist[pl.ds(sh * _NBINS + cc * 16, 16)]
            rchunk = lax.rev(chunk, (0,))
            sfx = plsc.cumsum(rchunk) + cum
            ge = sfx >= _TOP_K
            cnt_ge = jnp.sum(ge.astype(jnp.int32))
            pos = jnp.int32(16) - cnt_ge
            s_pos = jnp.min(jnp.where(ge, sfx, jnp.int32(_BIG)))
            prev = jnp.maximum(jnp.max(jnp.where(ge, jnp.int32(-_BIG), sfx)),
                               cum)
            hit = jnp.logical_and(jnp.logical_not(found), cnt_ge > 0)
            return (jnp.logical_or(found, cnt_ge > 0),
                    jnp.where(hit, cc * 16 + 15 - pos, bin_b),
                    jnp.where(hit, prev, c_gt),
                    jnp.where(hit, s_pos - prev, n_b),
                    cum + jnp.sum(rchunk))
        _, bin_b, c_gt, n_b, _ = lax.fori_loop(
            0, _NBINS // 16, scan_body,
            (jnp.bool_(False), jnp.int32(0), jnp.int32(0), jnp.int32(0),
             jnp.int32(0)))

        lf_v = (lo + (jnp.broadcast_to(bin_b, (16,)).astype(jnp.float32)
                      - 2.0) / scale)

        def col_window(w, carry):
            hit = jnp.any(wmax[pl.ds(w * 16, 16)] >= lf_v)

            def do(carry):
                cnt_hi, cnt_in = carry
                for u in range(_WCHUNKS):
                    c = w * _WCHUNKS + u
                    v = row_v[pl.ds(c * 16, 16)]
                    binv = to_bin(v)
                    mhi = binv > bin_b
                    min_ = jnp.logical_and(binv == bin_b, cnt_in < _CAP_IN)
                    nhi = jnp.sum(mhi.astype(jnp.int32))
                    nin = jnp.sum(min_.astype(jnp.int32))

                    @pl.when(nhi + nin > 0)
                    def _store(cnt_hi=cnt_hi, cnt_in=cnt_in, v=v,
                               mhi=mhi, min_=min_, c=c):
                        idxv = c * 16 + iota
                        plsc.store_compressed(buf_v.at[pl.ds(cnt_hi, 16)],
                                              v, mask=mhi)
                        plsc.store_compressed(buf_i.at[pl.ds(cnt_hi, 16)],
                                              idxv, mask=mhi)
                        plsc.store_compressed(
                            buf_v.at[pl.ds(_HI_REGION + cnt_in, 16)],
                            v, mask=min_)
                        plsc.store_compressed(
                            buf_i.at[pl.ds(_HI_REGION + cnt_in, 16)],
                            idxv, mask=min_)
                    cnt_hi = cnt_hi + nhi
                    cnt_in = cnt_in + nin
                return (cnt_hi, cnt_in)

            return lax.cond(hit, do, lambda c_: c_, carry)
        lax.fori_loop(0, _NWIN, col_window, (jnp.int32(0), jnp.int32(0)))

        @pl.when(r < _ROWS_PER_W - 1)
        def _prefetch():
            pltpu.async_copy(logits_hbm.at[row + 1], row_v, dma_sem)

        def sel_body(i, _c):
            bv, bp = neginf, jnp.full((16,), _BIG, jnp.int32)
            for j in range(_NVREG):
                x = buf_v[pl.ds(j * 16, 16)]
                m = x > bv
                bv = jnp.where(m, x, bv)
                bp = jnp.where(m, j * 16 + iota, bp)
            mx = jnp.max(bv)
            p = jnp.min(jnp.where(bv == mx, bp, jnp.int32(_BIG)))
            pv = jnp.broadcast_to(p, (16,))
            idx_p = jnp.max(plsc.load_gather(buf_i, [pv]))
            plsc.store_scatter(buf_v, [pv], neginf, mask=lane0)
            iv = jnp.broadcast_to(i, (16,))
            plsc.store_scatter(out_v, [iv], jnp.broadcast_to(mx, (16,)),
                               mask=lane0)
            plsc.store_scatter(out_i, [iv], jnp.broadcast_to(idx_p, (16,)),
                               mask=lane0)
            return 0
        lax.fori_loop(0, _TOP_K, sel_body, 0)

        vals = [out_v[pl.ds(j * 16, 16)] for j in range(4)]
        vmax = jnp.max(vals[0])
        es = [jnp.exp(v - vmax) for v in vals]
        s = es[0] + es[1] + es[2] + es[3]
        total = jnp.sum(s)
        keeps, masked = [], []
        carry = jnp.float32(0.0)
        for j in range(4):
            pj = es[j] / total
            cj = plsc.cumsum(pj) + carry
            carry = jnp.max(cj)
            kp = cj <= _TOP_P
            if j == 0:
                kp = jnp.logical_or(kp, lane0)
            keeps.append(kp)
            masked.append(jnp.where(kp, vals[j], -jnp.inf)
                          + kinf_v[pl.ds(j * 16, 16)])
        bm, bp = neginf, jnp.full((16,), _BIG, jnp.int32)
        for j in range(4):
            sc = masked[j] + g_v[pl.ds(j * 16, 16)]
            m = sc > bm
            bm = jnp.where(m, sc, bm)
            bp = jnp.where(m, j * 16 + iota, bp)
        mx2 = jnp.max(bm)
        p2 = jnp.min(jnp.where(bm == mx2, bp, jnp.int32(_BIG)))
        token = jnp.max(plsc.load_gather(out_i, [jnp.broadcast_to(p2, (16,))]))

        e2 = [jnp.where(jnp.logical_and(
                  keeps[j], kinf_v[pl.ds(j * 16, 16)] == 0.0),
                  es[j], 0.0) for j in range(4)]
        s2 = jnp.sum(e2[0] + e2[1] + e2[2] + e2[3])
        for j in range(4):
            probs_st[pl.ds(j * 16, 16)] = e2[j] / s2
        tok_st[...] = jnp.where(lane0, token, 0)

        pltpu.sync_copy(probs_st, probs_hbm.at[row])
        pltpu.sync_copy(tok_st, tok_hbm.at[row])
        return 0

    lax.fori_loop(0, _ROWS_PER_W, row_body, 0)


@functools.partial(jax.jit, static_argnames=())
def _sc_topk_sample(logits, gumbel, kinf):
    mesh = plsc.VectorSubcoreMesh(core_axis_name="c", subcore_axis_name="s")
    f = pl.kernel(
        _sc_body,
        mesh=mesh,
        compiler_params=pltpu.CompilerParams(needs_layout_passes=False),
        out_type=(
            jax.ShapeDtypeStruct((_R, 16), jnp.int32),
            jax.ShapeDtypeStruct((_R, _TOP_K), jnp.float32),
        ),
        scratch_types=[
            pltpu.VMEM((_V,), jnp.float32),
            pltpu.VMEM((_HIST_SLOTS,), jnp.int32),
            pltpu.VMEM((_NWIN * 16,), jnp.float32),
            pltpu.VMEM((_BUF,), jnp.float32),
            pltpu.VMEM((_BUF,), jnp.int32),
            pltpu.VMEM((_TOP_K,), jnp.float32),
            pltpu.VMEM((_TOP_K,), jnp.int32),
            pltpu.VMEM((_TOP_K,), jnp.float32),
            pltpu.VMEM((_TOP_K,), jnp.float32),
            pltpu.VMEM((_TOP_K,), jnp.float32),
            pltpu.VMEM((16,), jnp.int32),
            pltpu.SemaphoreType.DMA,
        ],
    )
    return f(logits, gumbel, kinf)


def kernel(logits, k):
    gumbel = jax.random.gumbel(jax.random.key(42), (_R, _TOP_K), jnp.float32)
    kinf = jnp.where(jnp.arange(_TOP_K) < k, 0.0, -jnp.inf).astype(jnp.float32)
    tok, probs = _sc_topk_sample(logits, gumbel, kinf)
    return tok[:, 0], probs
